# baseline (speedup 1.0000x reference)
import jax
import jax.numpy as jnp
from jax.experimental import pallas as pl
from jax.experimental.pallas import tpu as pltpu

_NUM_CHUNKS = 8


def _dma_copy(x_hbm, o_hbm, sems):
    rows = x_hbm.shape[0]
    chunk = rows // _NUM_CHUNKS
    for i in range(_NUM_CHUNKS):
        pltpu.make_async_copy(
            x_hbm.at[pl.ds(i * chunk, chunk), :],
            o_hbm.at[pl.ds(i * chunk, chunk), :],
            sems.at[i],
        ).start()
    for i in range(_NUM_CHUNKS):
        pltpu.make_async_copy(
            x_hbm.at[pl.ds(i * chunk, chunk), :],
            o_hbm.at[pl.ds(i * chunk, chunk), :],
            sems.at[i],
        ).wait()


def kernel(schedule_context):
    rows, cols = schedule_context.shape
    return pl.pallas_call(
        _dma_copy,
        in_specs=[pl.BlockSpec(memory_space=pl.ANY)],
        out_specs=pl.BlockSpec(memory_space=pl.ANY),
        out_shape=jax.ShapeDtypeStruct((rows, cols), schedule_context.dtype),
        scratch_shapes=[pltpu.SemaphoreType.DMA((_NUM_CHUNKS,))],
    )(schedule_context)


# speedup vs baseline: 49.0292x; 49.0292x over previous
import jax
import jax.numpy as jnp
from jax.experimental import pallas as pl
from jax.experimental.pallas import tpu as pltpu


def _copy_block(x_ref, o_ref):
    o_ref[...] = x_ref[...]


def kernel(schedule_context):
    rows, cols = schedule_context.shape
    block_rows = 1024
    return pl.pallas_call(
        _copy_block,
        grid=(rows // block_rows,),
        in_specs=[pl.BlockSpec((block_rows, cols), lambda i: (i, 0))],
        out_specs=pl.BlockSpec((block_rows, cols), lambda i: (i, 0)),
        out_shape=jax.ShapeDtypeStruct((rows, cols), schedule_context.dtype),
        compiler_params=pltpu.CompilerParams(
            dimension_semantics=("parallel",),
        ),
    )(schedule_context)
